# baseline (speedup 1.0000x reference)
import functools
import math

import numpy as np

import jax
import jax.numpy as jnp
from jax import lax
from jax.experimental import pallas as pl
from jax.experimental.pallas import tpu as pltpu
from jax.experimental.pallas import tpu_sc as plsc

VOCAB = 1000000
EMBED = 64
NUM_SAMPLED = 8192
BATCH = 4096

_NC = 2
_NS = 16
_NW = _NC * _NS
_TRUE_PER_W = BATCH // _NW
_SAMP_PER_W = NUM_SAMPLED // _NW
_K = 8
_G = 16

_LOGV1 = math.log(float(VOCAB) + 1.0)


@functools.lru_cache(maxsize=1)
def _samp_plan():
    def _sample_ids():
        skey = jax.random.key(42)
        u = jax.random.uniform(skey, (NUM_SAMPLED,), dtype=jnp.float32)
        return jnp.clip(
            (jnp.exp(u * jnp.log(VOCAB + 1.0)) - 1.0).astype(jnp.int32),
            0, VOCAB - 1)

    cpu = jax.devices("cpu")[0]
    with jax.default_device(cpu):
        ids = np.asarray(jax.jit(_sample_ids)())

    sids = ids[np.argsort(ids, kind="stable")]
    blocks = sids >> 7
    runs = []
    i = 0
    while i < NUM_SAMPLED:
        j = i
        while j < NUM_SAMPLED and blocks[j] == blocks[i]:
            j += 1
        runs.append((int(blocks[i]), list(map(int, sids[i:j]))))
        i = j

    runs.sort(key=lambda r: len(r[1]))
    bins = [[] for _ in range(_NW)]
    cap = [_SAMP_PER_W] * _NW
    stp = [0] * _NW
    t_est = math.ceil(sum(math.ceil(len(r[1]) / _G) for r in runs) / _NW) + 2
    for b, rids in runs:
        while rids:
            avail = [x for x in range(_NW) if cap[x] > 0]
            w = min(avail, key=lambda x: (stp[x], -cap[x]))
            room = max(_G * (t_est - stp[w]), _G)
            take = min(len(rids), cap[w], room)
            bins[w].append((b, rids[:take]))
            stp[w] += math.ceil(take / _G)
            rids = rids[take:]
            cap[w] -= take

    steps = []
    for pieces in bins:
        st = []
        for b, rids in pieces:
            for c in range(0, len(rids), _G):
                st.append((b, rids[c:c + _G]))
        steps.append(st)
    s_max = max(len(s) for s in steps)
    s_max = ((s_max + _K - 1) // _K) * _K

    fid = np.zeros((NUM_SAMPLED,), np.int32)
    sblk = np.zeros((_NW, s_max), np.int32)
    lc = np.zeros((_NW, s_max, _G), np.int32)
    for w in range(_NW):
        col = 0
        last = None
        for s in range(s_max):
            if s < len(steps[w]):
                b, rids = steps[w][s]
                ext = []
                for rid in rids:
                    assert rid >> 7 == b
                    fid[w * _SAMP_PER_W + col] = rid
                    ext.append(((rid & 127) << 8) | col)
                    col += 1
                while len(ext) < _G:
                    ext.append(ext[-1])
                last = (b, ext)
            b, ext = last
            sblk[w, s] = b
            lc[w, s, :] = ext
        assert col == _SAMP_PER_W
    return (fid, sblk.reshape(-1), lc.reshape(-1), s_max)


_PLAN = _samp_plan()


def _rd(lane, vec_ref, j):
    vec = vec_ref[pl.ds((j >> 4) * 16, 16)]
    return jnp.sum(jnp.where(lane == (j & 15), vec, 0))


def _fire_blk(wT_hbm, ring_v, sems, blk, slot):
    off = pl.multiple_of(blk * 128, 128)
    pltpu.async_copy(wT_hbm.at[:, pl.ds(off, 128)], ring_v.at[slot],
                     sems[slot])


def _drain(wT_hbm, ring_v, sems, slot):
    pltpu.make_async_copy(wT_hbm.at[:, pl.ds(0, 128)], ring_v.at[slot],
                          sems[slot]).wait()


def _extract(lane, ring_v, dst_v, lrow, col, slot):
    lcol = jnp.full((16,), 0, jnp.int32) + lrow
    jcol = jnp.full((16,), 0, jnp.int32) + col
    for q in range(EMBED // 16):
        rows = lane + q * 16
        v = plsc.load_gather(ring_v.at[slot], [rows, lcol])
        plsc.store_scatter(dst_v, [rows, jcol], v)


def _sc_gather_samp(wT, biases, sfid, sblk, slc, s_max):
    mesh = plsc.VectorSubcoreMesh(core_axis_name="c", subcore_axis_name="s")

    @functools.partial(
        pl.kernel,
        mesh=mesh,
        out_type=[
            jax.ShapeDtypeStruct((EMBED, NUM_SAMPLED), jnp.float32),
            jax.ShapeDtypeStruct((NUM_SAMPLED,), jnp.float32),
        ],
        scratch_types=[
            pltpu.VMEM((_SAMP_PER_W,), jnp.int32),
            pltpu.VMEM((s_max,), jnp.int32),
            pltpu.VMEM((s_max * _G,), jnp.int32),
            pltpu.VMEM((_K, EMBED, 128), jnp.float32),
            pltpu.VMEM((EMBED, _SAMP_PER_W), jnp.float32),
            pltpu.VMEM((_SAMP_PER_W,), jnp.float32),
        ] + [pltpu.SemaphoreType.DMA] * (_K + 1),
        compiler_params=pltpu.CompilerParams(
            use_tc_tiling_on_sc=True, needs_layout_passes=False),
    )
    def samp_kernel(wT_hbm, b_hbm, sfid_hbm, sblk_hbm, slc_hbm,
                    swT_out, sb_out,
                    sfid_v, sblk_v, slc_v, ring_v, scol_v, sb_v,
                    *all_sems):
        sems = all_sems[:_K]
        semb = all_sems[_K]
        lane = lax.iota(jnp.int32, 16)
        wid = lax.axis_index("s") * _NC + lax.axis_index("c")
        bs = wid * _SAMP_PER_W

        pltpu.sync_copy(sfid_hbm.at[pl.ds(bs, _SAMP_PER_W)], sfid_v)
        pltpu.sync_copy(sblk_hbm.at[pl.ds(wid * s_max, s_max)], sblk_v)
        pltpu.sync_copy(slc_hbm.at[pl.ds(wid * s_max * _G, s_max * _G)],
                        slc_v)
        cb = pltpu.async_copy(b_hbm.at[sfid_v], sb_v, semb)

        for b in range(_K):
            _fire_blk(wT_hbm, ring_v, sems, _rd(lane, sblk_v, b), b)

        def sgroup(g, carry):
            for b in range(_K):
                s = g * _K + b
                _drain(wT_hbm, ring_v, sems, b)
                for t in range(_G):
                    e = _rd(lane, slc_v, s * _G + t)
                    _extract(lane, ring_v, scol_v, e >> 8, e & 255, b)
                sn = s + _K

                @pl.when(sn < s_max)
                def _refire():
                    _fire_blk(wT_hbm, ring_v, sems,
                              _rd(lane, sblk_v, sn), b)
            return carry

        lax.fori_loop(0, s_max // _K, sgroup, 0, unroll=False)

        cb.wait()
        pltpu.sync_copy(scol_v, swT_out.at[:, pl.ds(bs, _SAMP_PER_W)])
        pltpu.sync_copy(sb_v, sb_out.at[pl.ds(bs, _SAMP_PER_W)])

    return samp_kernel(wT, biases, sfid, sblk, slc)


def _sc_gather_true(wT, biases, lbl_ids):
    mesh = plsc.VectorSubcoreMesh(core_axis_name="c", subcore_axis_name="s")

    @functools.partial(
        pl.kernel,
        mesh=mesh,
        out_type=[
            jax.ShapeDtypeStruct((EMBED, BATCH), jnp.float32),
            jax.ShapeDtypeStruct((BATCH,), jnp.float32),
        ],
        scratch_types=[
            pltpu.VMEM((_TRUE_PER_W,), jnp.int32),
            pltpu.VMEM((_K, EMBED, 128), jnp.float32),
            pltpu.VMEM((EMBED, _TRUE_PER_W), jnp.float32),
            pltpu.VMEM((_TRUE_PER_W,), jnp.float32),
        ] + [pltpu.SemaphoreType.DMA] * (_K + 1),
        compiler_params=pltpu.CompilerParams(
            use_tc_tiling_on_sc=True, needs_layout_passes=False),
    )
    def true_kernel(wT_hbm, b_hbm, li_hbm,
                    twT_out, tb_out,
                    li_v, ring_v, tcol_v, tb_v,
                    *all_sems):
        sems = all_sems[:_K]
        semb = all_sems[_K]
        lane = lax.iota(jnp.int32, 16)
        wid = lax.axis_index("s") * _NC + lax.axis_index("c")
        bt = wid * _TRUE_PER_W

        pltpu.sync_copy(li_hbm.at[pl.ds(bt, _TRUE_PER_W)], li_v)
        cb = pltpu.async_copy(b_hbm.at[li_v], tb_v, semb)

        def fire_lbl(j, slot):
            _fire_blk(wT_hbm, ring_v, sems, _rd(lane, li_v, j) >> 7, slot)

        for b in range(_K):
            fire_lbl(b, b)

        def lgroup(g, carry):
            for b in range(_K):
                j = g * _K + b
                _drain(wT_hbm, ring_v, sems, b)
                idx = _rd(lane, li_v, j)
                _extract(lane, ring_v, tcol_v, idx & 127, j, b)
                jn = j + _K

                @pl.when(jn < _TRUE_PER_W)
                def _refire():
                    fire_lbl(jn, b)
            return carry

        lax.fori_loop(0, _TRUE_PER_W // _K, lgroup, 0, unroll=False)

        cb.wait()
        pltpu.sync_copy(tcol_v, twT_out.at[:, pl.ds(bt, _TRUE_PER_W)])
        pltpu.sync_copy(tb_v, tb_out.at[pl.ds(bt, _TRUE_PER_W)])

    return true_kernel(wT, biases, lbl_ids)


_BB = 1024
_NBLK = BATCH // _BB


def _tc_samp_body(predT_ref, swT_ref, sshift_ref, m_ref, se_ref):
    predT = predT_ref[...]
    swT = swT_ref[...]
    logitsT = lax.dot_general(
        swT, predT, (((0,), (0,)), ((), ())),
        preferred_element_type=jnp.float32)
    logitsT = logitsT + sshift_ref[...]
    m = jnp.max(logitsT, axis=0, keepdims=True)
    se = jnp.sum(jnp.exp(logitsT - m), axis=0, keepdims=True)
    m_ref[0] = m
    se_ref[0] = se


def _tc_samp(predT, samp_wT, samp_shift):
    return pl.pallas_call(
        _tc_samp_body,
        grid=(_NBLK,),
        in_specs=[
            pl.BlockSpec((EMBED, _BB), lambda i: (0, i)),
            pl.BlockSpec((EMBED, NUM_SAMPLED), lambda i: (0, 0)),
            pl.BlockSpec((NUM_SAMPLED, 1), lambda i: (0, 0)),
        ],
        out_specs=[
            pl.BlockSpec((1, 1, _BB), lambda i: (i, 0, 0)),
            pl.BlockSpec((1, 1, _BB), lambda i: (i, 0, 0)),
        ],
        out_shape=[
            jax.ShapeDtypeStruct((_NBLK, 1, _BB), jnp.float32),
            jax.ShapeDtypeStruct((_NBLK, 1, _BB), jnp.float32),
        ],
        compiler_params=pltpu.CompilerParams(
            dimension_semantics=("arbitrary",)),
    )(predT, samp_wT, samp_shift)


def _tc_final_body(predT_ref, twT_ref, tb_ref, lbl_ref, m_ref, se_ref,
                   out_ref):
    i = pl.program_id(0)
    predT = predT_ref[...]
    twT = twT_ref[...]
    tlogit = jnp.sum(predT * twT, axis=0, keepdims=True) + tb_ref[0]
    lbl = lbl_ref[0]
    lblf = lbl.astype(jnp.float32)
    p = jnp.log((lblf + 2.0) / (lblf + 1.0)) * (1.0 / _LOGV1)
    log1p_neg = -p * (1.0 + p * (1.0 / 2.0 + p * (1.0 / 3.0 + p * (
        1.0 / 4.0 + p * (1.0 / 5.0 + p * (1.0 / 6.0 + p * (1.0 / 7.0)))))))
    x = NUM_SAMPLED * log1p_neg
    xs = jnp.maximum(x, -0.5)
    em1_series = xs * (1.0 + xs * (1.0 / 2.0 + xs * (1.0 / 6.0 + xs * (
        1.0 / 24.0 + xs * (1.0 / 120.0 + xs * (1.0 / 720.0 + xs * (
            1.0 / 5040.0)))))))
    em1 = jnp.where(x < -0.5, jnp.exp(x) - 1.0, em1_series)
    tlogit = tlogit - jnp.log(-em1)

    ms = m_ref[0]
    ses = se_ref[0]
    mt = jnp.maximum(ms, tlogit)
    se = ses * jnp.exp(ms - mt) + jnp.exp(tlogit - mt)
    per_ex = mt + jnp.log(se) - tlogit

    @pl.when(i == 0)
    def _init():
        out_ref[...] = jnp.zeros_like(out_ref)

    out_ref[...] += jnp.sum(per_ex) * (1.0 / BATCH)


def _tc_final(predT, true_wT, true_b3, labels3, m3, se3):
    return pl.pallas_call(
        _tc_final_body,
        grid=(_NBLK,),
        in_specs=[
            pl.BlockSpec((EMBED, _BB), lambda i: (0, i)),
            pl.BlockSpec((EMBED, _BB), lambda i: (0, i)),
            pl.BlockSpec((1, 1, _BB), lambda i: (i, 0, 0)),
            pl.BlockSpec((1, 1, _BB), lambda i: (i, 0, 0)),
            pl.BlockSpec((1, 1, _BB), lambda i: (i, 0, 0)),
            pl.BlockSpec((1, 1, _BB), lambda i: (i, 0, 0)),
        ],
        out_specs=pl.BlockSpec((1, 1), lambda i: (0, 0)),
        out_shape=jax.ShapeDtypeStruct((1, 1), jnp.float32),
        compiler_params=pltpu.CompilerParams(
            dimension_semantics=("arbitrary",)),
    )(predT, true_wT, true_b3, labels3, m3, se3)


def kernel(predictions, labels, weights, biases):
    labels_flat = labels.reshape(-1).astype(jnp.int32)

    fid_np, sblk_np, slc_np, s_max = _PLAN
    fid = jnp.asarray(fid_np)
    wT = weights.T
    predT = predictions.T

    samp_wT, samp_b = _sc_gather_samp(
        wT, biases, fid, jnp.asarray(sblk_np), jnp.asarray(slc_np), s_max)

    labels_seq = lax.optimization_barrier((labels_flat, samp_wT))[0]
    true_wT, true_b = _sc_gather_true(wT, biases, labels_seq)

    sampf = fid.astype(jnp.float32)
    p_samp = jnp.log((sampf + 2.0) / (sampf + 1.0)) / _LOGV1
    samp_expected = -jnp.expm1(NUM_SAMPLED * jnp.log1p(-p_samp))
    samp_shift = (samp_b - jnp.log(samp_expected)).reshape(NUM_SAMPLED, 1)

    m3, se3 = _tc_samp(predT, samp_wT, samp_shift)

    loss = _tc_final(predT, true_wT,
                     true_b.reshape(_NBLK, 1, _BB),
                     labels_flat.reshape(_NBLK, 1, _BB),
                     m3, se3)
    return loss[0, 0]


# speedup vs baseline: 1.1749x; 1.1749x over previous
import functools
import math

import numpy as np

import jax
import jax.numpy as jnp
from jax import lax
from jax.experimental import pallas as pl
from jax.experimental.pallas import tpu as pltpu
from jax.experimental.pallas import tpu_sc as plsc

VOCAB = 1000000
EMBED = 64
NUM_SAMPLED = 8192
BATCH = 4096

_NC = 2
_NS = 16
_NW = _NC * _NS
_TRUE_PER_W = BATCH // _NW
_SAMP_PER_W = NUM_SAMPLED // _NW
_K = 8
_G = 8

_LOGV1 = math.log(float(VOCAB) + 1.0)


@functools.lru_cache(maxsize=1)
def _samp_plan():
    def _sample_ids():
        skey = jax.random.key(42)
        u = jax.random.uniform(skey, (NUM_SAMPLED,), dtype=jnp.float32)
        return jnp.clip(
            (jnp.exp(u * jnp.log(VOCAB + 1.0)) - 1.0).astype(jnp.int32),
            0, VOCAB - 1)

    cpu = jax.devices("cpu")[0]
    with jax.default_device(cpu):
        ids = np.asarray(jax.jit(_sample_ids)())

    sids = ids[np.argsort(ids, kind="stable")]
    blocks = sids >> 7
    runs = []
    i = 0
    while i < NUM_SAMPLED:
        j = i
        while j < NUM_SAMPLED and blocks[j] == blocks[i]:
            j += 1
        runs.append((int(blocks[i]), list(map(int, sids[i:j]))))
        i = j

    runs.sort(key=lambda r: len(r[1]))
    bins = [[] for _ in range(_NW)]
    cap = [_SAMP_PER_W] * _NW
    stp = [0] * _NW
    t_est = math.ceil(sum(math.ceil(len(r[1]) / _G) for r in runs) / _NW) + 2
    for b, rids in runs:
        while rids:
            avail = [x for x in range(_NW) if cap[x] > 0]
            w = min(avail, key=lambda x: (stp[x], -cap[x]))
            room = max(_G * (t_est - stp[w]), _G)
            take = min(len(rids), cap[w], room)
            bins[w].append((b, rids[:take]))
            stp[w] += math.ceil(take / _G)
            rids = rids[take:]
            cap[w] -= take

    steps = []
    for pieces in bins:
        st = []
        for b, rids in pieces:
            for c in range(0, len(rids), _G):
                st.append((b, rids[c:c + _G]))
        steps.append(st)
    s_max = max(len(s) for s in steps)
    s_max = ((s_max + _K - 1) // _K) * _K

    fid = np.zeros((NUM_SAMPLED,), np.int32)
    sblk = np.zeros((_NW, s_max), np.int32)
    lc = np.zeros((_NW, s_max, _G), np.int32)
    for w in range(_NW):
        col = 0
        last = None
        for s in range(s_max):
            if s < len(steps[w]):
                b, rids = steps[w][s]
                ext = []
                for rid in rids:
                    assert rid >> 7 == b
                    fid[w * _SAMP_PER_W + col] = rid
                    ext.append(((rid & 127) << 8) | col)
                    col += 1
                while len(ext) < _G:
                    ext.append(ext[-1])
                last = (b, ext)
            b, ext = last
            sblk[w, s] = b
            lc[w, s, :] = ext
        assert col == _SAMP_PER_W
    return (fid, sblk.reshape(-1), lc.reshape(-1), s_max)


_PLAN = _samp_plan()


def _rd(lane, vec_ref, j):
    vec = vec_ref[pl.ds((j >> 4) * 16, 16)]
    return jnp.sum(jnp.where(lane == (j & 15), vec, 0))


def _fire_blk(wT_hbm, ring_v, sems, blk, slot):
    off = pl.multiple_of(blk * 128, 128)
    pltpu.async_copy(wT_hbm.at[:, pl.ds(off, 128)], ring_v.at[slot],
                     sems[slot])


def _drain(wT_hbm, ring_v, sems, slot):
    pltpu.make_async_copy(wT_hbm.at[:, pl.ds(0, 128)], ring_v.at[slot],
                          sems[slot]).wait()


def _extract(lane, ring_v, dst_v, lrow, col, slot):
    lcol = jnp.full((16,), 0, jnp.int32) + lrow
    jcol = jnp.full((16,), 0, jnp.int32) + col
    for q in range(EMBED // 16):
        rows = lane + q * 16
        v = plsc.load_gather(ring_v.at[slot], [rows, lcol])
        plsc.store_scatter(dst_v, [rows, jcol], v)


def _sc_gather_samp(wT, biases, sfid, sblk, slc, s_max):
    mesh = plsc.VectorSubcoreMesh(core_axis_name="c", subcore_axis_name="s")

    @functools.partial(
        pl.kernel,
        mesh=mesh,
        out_type=[
            jax.ShapeDtypeStruct((EMBED, NUM_SAMPLED), jnp.float32),
            jax.ShapeDtypeStruct((NUM_SAMPLED,), jnp.float32),
        ],
        scratch_types=[
            pltpu.VMEM((_SAMP_PER_W,), jnp.int32),
            pltpu.VMEM((s_max,), jnp.int32),
            pltpu.VMEM((s_max * _G,), jnp.int32),
            pltpu.VMEM((_K, EMBED, 128), jnp.float32),
            pltpu.VMEM((EMBED, _SAMP_PER_W), jnp.float32),
            pltpu.VMEM((_SAMP_PER_W,), jnp.float32),
        ] + [pltpu.SemaphoreType.DMA] * (_K + 1),
        compiler_params=pltpu.CompilerParams(
            use_tc_tiling_on_sc=True, needs_layout_passes=False),
    )
    def samp_kernel(wT_hbm, b_hbm, sfid_hbm, sblk_hbm, slc_hbm,
                    swT_out, sb_out,
                    sfid_v, sblk_v, slc_v, ring_v, scol_v, sb_v,
                    *all_sems):
        sems = all_sems[:_K]
        semb = all_sems[_K]
        lane = lax.iota(jnp.int32, 16)
        wid = lax.axis_index("s") * _NC + lax.axis_index("c")
        bs = wid * _SAMP_PER_W

        pltpu.sync_copy(sfid_hbm.at[pl.ds(bs, _SAMP_PER_W)], sfid_v)
        pltpu.sync_copy(sblk_hbm.at[pl.ds(wid * s_max, s_max)], sblk_v)
        pltpu.sync_copy(slc_hbm.at[pl.ds(wid * s_max * _G, s_max * _G)],
                        slc_v)
        cb = pltpu.async_copy(b_hbm.at[sfid_v], sb_v, semb)

        for b in range(_K):
            _fire_blk(wT_hbm, ring_v, sems, _rd(lane, sblk_v, b), b)

        def sgroup(g, carry):
            for b in range(_K):
                s = g * _K + b
                _drain(wT_hbm, ring_v, sems, b)
                for t in range(_G):
                    e = _rd(lane, slc_v, s * _G + t)
                    _extract(lane, ring_v, scol_v, e >> 8, e & 255, b)
                sn = s + _K

                @pl.when(sn < s_max)
                def _refire():
                    _fire_blk(wT_hbm, ring_v, sems,
                              _rd(lane, sblk_v, sn), b)
            return carry

        lax.fori_loop(0, s_max // _K, sgroup, 0, unroll=False)

        cb.wait()
        pltpu.sync_copy(scol_v, swT_out.at[:, pl.ds(bs, _SAMP_PER_W)])
        pltpu.sync_copy(sb_v, sb_out.at[pl.ds(bs, _SAMP_PER_W)])

    return samp_kernel(wT, biases, sfid, sblk, slc)


def _sc_gather_true(wT, biases, lbl_ids):
    mesh = plsc.VectorSubcoreMesh(core_axis_name="c", subcore_axis_name="s")

    @functools.partial(
        pl.kernel,
        mesh=mesh,
        out_type=[
            jax.ShapeDtypeStruct((EMBED, BATCH), jnp.float32),
            jax.ShapeDtypeStruct((BATCH,), jnp.float32),
        ],
        scratch_types=[
            pltpu.VMEM((_TRUE_PER_W,), jnp.int32),
            pltpu.VMEM((_K, EMBED, 128), jnp.float32),
            pltpu.VMEM((EMBED, _TRUE_PER_W), jnp.float32),
            pltpu.VMEM((_TRUE_PER_W,), jnp.float32),
        ] + [pltpu.SemaphoreType.DMA] * (_K + 1),
        compiler_params=pltpu.CompilerParams(
            use_tc_tiling_on_sc=True, needs_layout_passes=False),
    )
    def true_kernel(wT_hbm, b_hbm, li_hbm,
                    twT_out, tb_out,
                    li_v, ring_v, tcol_v, tb_v,
                    *all_sems):
        sems = all_sems[:_K]
        semb = all_sems[_K]
        lane = lax.iota(jnp.int32, 16)
        wid = lax.axis_index("s") * _NC + lax.axis_index("c")
        bt = wid * _TRUE_PER_W

        pltpu.sync_copy(li_hbm.at[pl.ds(bt, _TRUE_PER_W)], li_v)
        cb = pltpu.async_copy(b_hbm.at[li_v], tb_v, semb)

        def fire_lbl(j, slot):
            _fire_blk(wT_hbm, ring_v, sems, _rd(lane, li_v, j) >> 7, slot)

        for b in range(_K):
            fire_lbl(b, b)

        def lgroup(g, carry):
            for b in range(_K):
                j = g * _K + b
                _drain(wT_hbm, ring_v, sems, b)
                idx = _rd(lane, li_v, j)
                _extract(lane, ring_v, tcol_v, idx & 127, j, b)
                jn = j + _K

                @pl.when(jn < _TRUE_PER_W)
                def _refire():
                    fire_lbl(jn, b)
            return carry

        lax.fori_loop(0, _TRUE_PER_W // _K, lgroup, 0, unroll=False)

        cb.wait()
        pltpu.sync_copy(tcol_v, twT_out.at[:, pl.ds(bt, _TRUE_PER_W)])
        pltpu.sync_copy(tb_v, tb_out.at[pl.ds(bt, _TRUE_PER_W)])

    return true_kernel(wT, biases, lbl_ids)


_BB = 1024
_NBLK = BATCH // _BB


def _tc_samp_body(predT_ref, swT_ref, sshift_ref, m_ref, se_ref):
    predT = predT_ref[...]
    swT = swT_ref[...]
    logitsT = lax.dot_general(
        swT, predT, (((0,), (0,)), ((), ())),
        preferred_element_type=jnp.float32)
    logitsT = logitsT + sshift_ref[...]
    m = jnp.max(logitsT, axis=0, keepdims=True)
    se = jnp.sum(jnp.exp(logitsT - m), axis=0, keepdims=True)
    m_ref[0] = m
    se_ref[0] = se


def _tc_samp(predT, samp_wT, samp_shift):
    return pl.pallas_call(
        _tc_samp_body,
        grid=(_NBLK,),
        in_specs=[
            pl.BlockSpec((EMBED, _BB), lambda i: (0, i)),
            pl.BlockSpec((EMBED, NUM_SAMPLED), lambda i: (0, 0)),
            pl.BlockSpec((NUM_SAMPLED, 1), lambda i: (0, 0)),
        ],
        out_specs=[
            pl.BlockSpec((1, 1, _BB), lambda i: (i, 0, 0)),
            pl.BlockSpec((1, 1, _BB), lambda i: (i, 0, 0)),
        ],
        out_shape=[
            jax.ShapeDtypeStruct((_NBLK, 1, _BB), jnp.float32),
            jax.ShapeDtypeStruct((_NBLK, 1, _BB), jnp.float32),
        ],
        compiler_params=pltpu.CompilerParams(
            dimension_semantics=("arbitrary",)),
    )(predT, samp_wT, samp_shift)


def _tc_final_body(predT_ref, twT_ref, tb_ref, lbl_ref, m_ref, se_ref,
                   out_ref):
    i = pl.program_id(0)
    predT = predT_ref[...]
    twT = twT_ref[...]
    tlogit = jnp.sum(predT * twT, axis=0, keepdims=True) + tb_ref[0]
    lbl = lbl_ref[0]
    lblf = lbl.astype(jnp.float32)
    p = jnp.log((lblf + 2.0) / (lblf + 1.0)) * (1.0 / _LOGV1)
    log1p_neg = -p * (1.0 + p * (1.0 / 2.0 + p * (1.0 / 3.0 + p * (
        1.0 / 4.0 + p * (1.0 / 5.0 + p * (1.0 / 6.0 + p * (1.0 / 7.0)))))))
    x = NUM_SAMPLED * log1p_neg
    xs = jnp.maximum(x, -0.5)
    em1_series = xs * (1.0 + xs * (1.0 / 2.0 + xs * (1.0 / 6.0 + xs * (
        1.0 / 24.0 + xs * (1.0 / 120.0 + xs * (1.0 / 720.0 + xs * (
            1.0 / 5040.0)))))))
    em1 = jnp.where(x < -0.5, jnp.exp(x) - 1.0, em1_series)
    tlogit = tlogit - jnp.log(-em1)

    ms = m_ref[0]
    ses = se_ref[0]
    mt = jnp.maximum(ms, tlogit)
    se = ses * jnp.exp(ms - mt) + jnp.exp(tlogit - mt)
    per_ex = mt + jnp.log(se) - tlogit

    @pl.when(i == 0)
    def _init():
        out_ref[...] = jnp.zeros_like(out_ref)

    out_ref[...] += jnp.sum(per_ex) * (1.0 / BATCH)


def _tc_final(predT, true_wT, true_b3, labels3, m3, se3):
    return pl.pallas_call(
        _tc_final_body,
        grid=(_NBLK,),
        in_specs=[
            pl.BlockSpec((EMBED, _BB), lambda i: (0, i)),
            pl.BlockSpec((EMBED, _BB), lambda i: (0, i)),
            pl.BlockSpec((1, 1, _BB), lambda i: (i, 0, 0)),
            pl.BlockSpec((1, 1, _BB), lambda i: (i, 0, 0)),
            pl.BlockSpec((1, 1, _BB), lambda i: (i, 0, 0)),
            pl.BlockSpec((1, 1, _BB), lambda i: (i, 0, 0)),
        ],
        out_specs=pl.BlockSpec((1, 1), lambda i: (0, 0)),
        out_shape=jax.ShapeDtypeStruct((1, 1), jnp.float32),
        compiler_params=pltpu.CompilerParams(
            dimension_semantics=("arbitrary",)),
    )(predT, true_wT, true_b3, labels3, m3, se3)


def kernel(predictions, labels, weights, biases):
    labels_flat = labels.reshape(-1).astype(jnp.int32)

    fid_np, sblk_np, slc_np, s_max = _PLAN
    fid = jnp.asarray(fid_np)
    wT = weights.T
    predT = predictions.T

    samp_wT, samp_b = _sc_gather_samp(
        wT, biases, fid, jnp.asarray(sblk_np), jnp.asarray(slc_np), s_max)

    labels_seq = lax.optimization_barrier((labels_flat, samp_wT))[0]
    true_wT, true_b = _sc_gather_true(wT, biases, labels_seq)

    sampf = fid.astype(jnp.float32)
    p_samp = jnp.log((sampf + 2.0) / (sampf + 1.0)) / _LOGV1
    samp_expected = -jnp.expm1(NUM_SAMPLED * jnp.log1p(-p_samp))
    samp_shift = (samp_b - jnp.log(samp_expected)).reshape(NUM_SAMPLED, 1)

    m3, se3 = _tc_samp(predT, samp_wT, samp_shift)

    loss = _tc_final(predT, true_wT,
                     true_b.reshape(_NBLK, 1, _BB),
                     labels_flat.reshape(_NBLK, 1, _BB),
                     m3, se3)
    return loss[0, 0]


# speedup vs baseline: 1.2832x; 1.0922x over previous
import functools
import math

import numpy as np

import jax
import jax.numpy as jnp
from jax import lax
from jax.experimental import pallas as pl
from jax.experimental.pallas import tpu as pltpu
from jax.experimental.pallas import tpu_sc as plsc

VOCAB = 1000000
EMBED = 64
NUM_SAMPLED = 8192
BATCH = 4096

_NC = 2
_NS = 16
_NW = _NC * _NS
_TRUE_PER_W = BATCH // _NW
_SAMP_PER_W = NUM_SAMPLED // _NW
_K = 8
_G = 8

_LOGV1 = math.log(float(VOCAB) + 1.0)


@functools.lru_cache(maxsize=1)
def _samp_plan():
    def _sample_ids():
        skey = jax.random.key(42)
        u = jax.random.uniform(skey, (NUM_SAMPLED,), dtype=jnp.float32)
        return jnp.clip(
            (jnp.exp(u * jnp.log(VOCAB + 1.0)) - 1.0).astype(jnp.int32),
            0, VOCAB - 1)

    cpu = jax.devices("cpu")[0]
    with jax.default_device(cpu):
        ids = np.asarray(jax.jit(_sample_ids)())

    sids = ids[np.argsort(ids, kind="stable")]
    blocks = sids >> 7
    runs = []
    i = 0
    while i < NUM_SAMPLED:
        j = i
        while j < NUM_SAMPLED and blocks[j] == blocks[i]:
            j += 1
        runs.append((int(blocks[i]), list(map(int, sids[i:j]))))
        i = j

    runs.sort(key=lambda r: len(r[1]))
    bins = [[] for _ in range(_NW)]
    cap = [_SAMP_PER_W] * _NW
    stp = [0] * _NW
    t_est = math.ceil(sum(math.ceil(len(r[1]) / _G) for r in runs) / _NW) + 2
    for b, rids in runs:
        while rids:
            avail = [x for x in range(_NW) if cap[x] > 0]
            w = min(avail, key=lambda x: (stp[x], -cap[x]))
            room = max(_G * (t_est - stp[w]), _G)
            take = min(len(rids), cap[w], room)
            bins[w].append((b, rids[:take]))
            stp[w] += math.ceil(take / _G)
            rids = rids[take:]
            cap[w] -= take

    steps = []
    for pieces in bins:
        st = []
        for b, rids in pieces:
            for c in range(0, len(rids), _G):
                st.append((b, rids[c:c + _G]))
        steps.append(st)
    s_max = max(len(s) for s in steps)
    s_max = ((s_max + _K - 1) // _K) * _K

    fid = np.zeros((NUM_SAMPLED,), np.int32)
    sblk = np.zeros((_NW, s_max), np.int32)
    lc = np.zeros((_NW, s_max, _G), np.int32)
    for w in range(_NW):
        col = 0
        last = None
        for s in range(s_max):
            if s < len(steps[w]):
                b, rids = steps[w][s]
                ext = []
                for rid in rids:
                    assert rid >> 7 == b
                    fid[w * _SAMP_PER_W + col] = rid
                    ext.append(((rid & 127) << 8) | col)
                    col += 1
                while len(ext) < _G:
                    ext.append(-1)
                last = b
            else:
                b, ext = last, [-1] * _G
            sblk[w, s] = b
            lc[w, s, :] = ext
        assert col == _SAMP_PER_W
    return (fid, sblk.reshape(-1), lc.reshape(-1), s_max)


_PLAN = _samp_plan()


def _rd(lane, vec_ref, j):
    vec = vec_ref[pl.ds((j >> 4) * 16, 16)]
    return jnp.sum(jnp.where(lane == (j & 15), vec, 0))


def _fire_blk(wT_hbm, ring_v, sems, blk, slot):
    off = pl.multiple_of(blk * 128, 128)
    pltpu.async_copy(wT_hbm.at[:, pl.ds(off, 128)], ring_v.at[slot],
                     sems[slot])


def _drain(wT_hbm, ring_v, sems, slot):
    pltpu.make_async_copy(wT_hbm.at[:, pl.ds(0, 128)], ring_v.at[slot],
                          sems[slot]).wait()


def _extract(lane, ring_v, dst_v, lrow, col, slot):
    lcol = jnp.full((16,), 0, jnp.int32) + lrow
    jcol = jnp.full((16,), 0, jnp.int32) + col
    for q in range(EMBED // 16):
        rows = lane + q * 16
        v = plsc.load_gather(ring_v.at[slot], [rows, lcol])
        plsc.store_scatter(dst_v, [rows, jcol], v)


def _sc_gather_samp(wT, biases, sfid, sblk, slc, s_max):
    mesh = plsc.VectorSubcoreMesh(core_axis_name="c", subcore_axis_name="s")

    @functools.partial(
        pl.kernel,
        mesh=mesh,
        out_type=[
            jax.ShapeDtypeStruct((EMBED, NUM_SAMPLED), jnp.float32),
            jax.ShapeDtypeStruct((NUM_SAMPLED,), jnp.float32),
        ],
        scratch_types=[
            pltpu.VMEM((_SAMP_PER_W,), jnp.int32),
            pltpu.VMEM((s_max,), jnp.int32),
            pltpu.VMEM((s_max * _G,), jnp.int32),
            pltpu.VMEM((_K, EMBED, 128), jnp.float32),
            pltpu.VMEM((EMBED, _SAMP_PER_W), jnp.float32),
            pltpu.VMEM((_SAMP_PER_W,), jnp.float32),
        ] + [pltpu.SemaphoreType.DMA] * (_K + 1),
        compiler_params=pltpu.CompilerParams(
            use_tc_tiling_on_sc=True, needs_layout_passes=False),
    )
    def samp_kernel(wT_hbm, b_hbm, sfid_hbm, sblk_hbm, slc_hbm,
                    swT_out, sb_out,
                    sfid_v, sblk_v, slc_v, ring_v, scol_v, sb_v,
                    *all_sems):
        sems = all_sems[:_K]
        semb = all_sems[_K]
        lane = lax.iota(jnp.int32, 16)
        wid = lax.axis_index("s") * _NC + lax.axis_index("c")
        bs = wid * _SAMP_PER_W

        pltpu.sync_copy(sfid_hbm.at[pl.ds(bs, _SAMP_PER_W)], sfid_v)
        pltpu.sync_copy(sblk_hbm.at[pl.ds(wid * s_max, s_max)], sblk_v)
        pltpu.sync_copy(slc_hbm.at[pl.ds(wid * s_max * _G, s_max * _G)],
                        slc_v)
        cb = pltpu.async_copy(b_hbm.at[sfid_v], sb_v, semb)

        for b in range(_K):
            _fire_blk(wT_hbm, ring_v, sems, _rd(lane, sblk_v, b), b)

        def sgroup(g, carry):
            for b in range(_K):
                s = g * _K + b
                _drain(wT_hbm, ring_v, sems, b)
                for t in range(_G):
                    e = _rd(lane, slc_v, s * _G + t)

                    @pl.when(e >= 0)
                    def _do_extract():
                        _extract(lane, ring_v, scol_v, e >> 8, e & 255, b)
                sn = s + _K

                @pl.when(sn < s_max)
                def _refire():
                    _fire_blk(wT_hbm, ring_v, sems,
                              _rd(lane, sblk_v, sn), b)
            return carry

        lax.fori_loop(0, s_max // _K, sgroup, 0, unroll=False)

        cb.wait()
        pltpu.sync_copy(scol_v, swT_out.at[:, pl.ds(bs, _SAMP_PER_W)])
        pltpu.sync_copy(sb_v, sb_out.at[pl.ds(bs, _SAMP_PER_W)])

    return samp_kernel(wT, biases, sfid, sblk, slc)


def _sc_gather_true(wT, biases, lbl_ids):
    mesh = plsc.VectorSubcoreMesh(core_axis_name="c", subcore_axis_name="s")

    @functools.partial(
        pl.kernel,
        mesh=mesh,
        out_type=[
            jax.ShapeDtypeStruct((EMBED, BATCH), jnp.float32),
            jax.ShapeDtypeStruct((BATCH,), jnp.float32),
        ],
        scratch_types=[
            pltpu.VMEM((_TRUE_PER_W,), jnp.int32),
            pltpu.VMEM((_K, EMBED, 128), jnp.float32),
            pltpu.VMEM((EMBED, _TRUE_PER_W), jnp.float32),
            pltpu.VMEM((_TRUE_PER_W,), jnp.float32),
        ] + [pltpu.SemaphoreType.DMA] * (_K + 1),
        compiler_params=pltpu.CompilerParams(
            use_tc_tiling_on_sc=True, needs_layout_passes=False),
    )
    def true_kernel(wT_hbm, b_hbm, li_hbm,
                    twT_out, tb_out,
                    li_v, ring_v, tcol_v, tb_v,
                    *all_sems):
        sems = all_sems[:_K]
        semb = all_sems[_K]
        lane = lax.iota(jnp.int32, 16)
        wid = lax.axis_index("s") * _NC + lax.axis_index("c")
        bt = wid * _TRUE_PER_W

        pltpu.sync_copy(li_hbm.at[pl.ds(bt, _TRUE_PER_W)], li_v)
        cb = pltpu.async_copy(b_hbm.at[li_v], tb_v, semb)

        def fire_lbl(j, slot):
            _fire_blk(wT_hbm, ring_v, sems, _rd(lane, li_v, j) >> 7, slot)

        for b in range(_K):
            fire_lbl(b, b)

        def lgroup(g, carry):
            for b in range(_K):
                j = g * _K + b
                _drain(wT_hbm, ring_v, sems, b)
                idx = _rd(lane, li_v, j)
                _extract(lane, ring_v, tcol_v, idx & 127, j, b)
                jn = j + _K

                @pl.when(jn < _TRUE_PER_W)
                def _refire():
                    fire_lbl(jn, b)
            return carry

        lax.fori_loop(0, _TRUE_PER_W // _K, lgroup, 0, unroll=False)

        cb.wait()
        pltpu.sync_copy(tcol_v, twT_out.at[:, pl.ds(bt, _TRUE_PER_W)])
        pltpu.sync_copy(tb_v, tb_out.at[pl.ds(bt, _TRUE_PER_W)])

    return true_kernel(wT, biases, lbl_ids)


_BB = 1024
_NBLK = BATCH // _BB


def _tc_samp_body(predT_ref, swT_ref, sshift_ref, m_ref, se_ref):
    predT = predT_ref[...]
    swT = swT_ref[...]
    logitsT = lax.dot_general(
        swT, predT, (((0,), (0,)), ((), ())),
        preferred_element_type=jnp.float32)
    logitsT = logitsT + sshift_ref[...]
    m = jnp.max(logitsT, axis=0, keepdims=True)
    se = jnp.sum(jnp.exp(logitsT - m), axis=0, keepdims=True)
    m_ref[0] = m
    se_ref[0] = se


def _tc_samp(predT, samp_wT, samp_shift):
    return pl.pallas_call(
        _tc_samp_body,
        grid=(_NBLK,),
        in_specs=[
            pl.BlockSpec((EMBED, _BB), lambda i: (0, i)),
            pl.BlockSpec((EMBED, NUM_SAMPLED), lambda i: (0, 0)),
            pl.BlockSpec((NUM_SAMPLED, 1), lambda i: (0, 0)),
        ],
        out_specs=[
            pl.BlockSpec((1, 1, _BB), lambda i: (i, 0, 0)),
            pl.BlockSpec((1, 1, _BB), lambda i: (i, 0, 0)),
        ],
        out_shape=[
            jax.ShapeDtypeStruct((_NBLK, 1, _BB), jnp.float32),
            jax.ShapeDtypeStruct((_NBLK, 1, _BB), jnp.float32),
        ],
        compiler_params=pltpu.CompilerParams(
            dimension_semantics=("arbitrary",)),
    )(predT, samp_wT, samp_shift)


def _tc_final_body(predT_ref, twT_ref, tb_ref, lbl_ref, m_ref, se_ref,
                   out_ref):
    i = pl.program_id(0)
    predT = predT_ref[...]
    twT = twT_ref[...]
    tlogit = jnp.sum(predT * twT, axis=0, keepdims=True) + tb_ref[0]
    lbl = lbl_ref[0]
    lblf = lbl.astype(jnp.float32)
    p = jnp.log((lblf + 2.0) / (lblf + 1.0)) * (1.0 / _LOGV1)
    log1p_neg = -p * (1.0 + p * (1.0 / 2.0 + p * (1.0 / 3.0 + p * (
        1.0 / 4.0 + p * (1.0 / 5.0 + p * (1.0 / 6.0 + p * (1.0 / 7.0)))))))
    x = NUM_SAMPLED * log1p_neg
    xs = jnp.maximum(x, -0.5)
    em1_series = xs * (1.0 + xs * (1.0 / 2.0 + xs * (1.0 / 6.0 + xs * (
        1.0 / 24.0 + xs * (1.0 / 120.0 + xs * (1.0 / 720.0 + xs * (
            1.0 / 5040.0)))))))
    em1 = jnp.where(x < -0.5, jnp.exp(x) - 1.0, em1_series)
    tlogit = tlogit - jnp.log(-em1)

    ms = m_ref[0]
    ses = se_ref[0]
    mt = jnp.maximum(ms, tlogit)
    se = ses * jnp.exp(ms - mt) + jnp.exp(tlogit - mt)
    per_ex = mt + jnp.log(se) - tlogit

    @pl.when(i == 0)
    def _init():
        out_ref[...] = jnp.zeros_like(out_ref)

    out_ref[...] += jnp.sum(per_ex) * (1.0 / BATCH)


def _tc_final(predT, true_wT, true_b3, labels3, m3, se3):
    return pl.pallas_call(
        _tc_final_body,
        grid=(_NBLK,),
        in_specs=[
            pl.BlockSpec((EMBED, _BB), lambda i: (0, i)),
            pl.BlockSpec((EMBED, _BB), lambda i: (0, i)),
            pl.BlockSpec((1, 1, _BB), lambda i: (i, 0, 0)),
            pl.BlockSpec((1, 1, _BB), lambda i: (i, 0, 0)),
            pl.BlockSpec((1, 1, _BB), lambda i: (i, 0, 0)),
            pl.BlockSpec((1, 1, _BB), lambda i: (i, 0, 0)),
        ],
        out_specs=pl.BlockSpec((1, 1), lambda i: (0, 0)),
        out_shape=jax.ShapeDtypeStruct((1, 1), jnp.float32),
        compiler_params=pltpu.CompilerParams(
            dimension_semantics=("arbitrary",)),
    )(predT, true_wT, true_b3, labels3, m3, se3)


def kernel(predictions, labels, weights, biases):
    labels_flat = labels.reshape(-1).astype(jnp.int32)

    fid_np, sblk_np, slc_np, s_max = _PLAN
    fid = jnp.asarray(fid_np)
    wT = weights.T
    predT = predictions.T

    samp_wT, samp_b = _sc_gather_samp(
        wT, biases, fid, jnp.asarray(sblk_np), jnp.asarray(slc_np), s_max)

    labels_seq = lax.optimization_barrier((labels_flat, samp_wT))[0]
    true_wT, true_b = _sc_gather_true(wT, biases, labels_seq)

    sampf = fid.astype(jnp.float32)
    p_samp = jnp.log((sampf + 2.0) / (sampf + 1.0)) / _LOGV1
    samp_expected = -jnp.expm1(NUM_SAMPLED * jnp.log1p(-p_samp))
    samp_shift = (samp_b - jnp.log(samp_expected)).reshape(NUM_SAMPLED, 1)

    m3, se3 = _tc_samp(predT, samp_wT, samp_shift)

    loss = _tc_final(predT, true_wT,
                     true_b.reshape(_NBLK, 1, _BB),
                     labels_flat.reshape(_NBLK, 1, _BB),
                     m3, se3)
    return loss[0, 0]
